# baseline (speedup 1.0000x reference)
import functools

import jax
import jax.numpy as jnp
from jax import lax
from jax.experimental import pallas as pl
from jax.experimental.pallas import tpu as pltpu
from jax.experimental.pallas import tpu_sc as plsc

N = 4096
D = 128
C = 100
CP = 112
CP2 = 128
NS = 16
PTS = N // NS
CPT = CP // NS
L = 16
DCH = D // L
WB_W = 1.0
MM_W = 0.1

_mesh = plsc.VectorSubcoreMesh(core_axis_name="c", subcore_axis_name="s",
                               num_cores=1)

_GATHER_DNUMS = lax.GatherDimensionNumbers(
    offset_dims=(), collapsed_slice_dims=(0,), start_index_map=(0,))


def _shuffle(a, perm):
    return lax.gather(a, perm[:, None], dimension_numbers=_GATHER_DNUMS,
                      slice_sizes=(1,),
                      mode=lax.GatherScatterMode.PROMISE_IN_BOUNDS)


def _vsum(a):
    lanes = lax.iota(jnp.int32, L)
    for sh in (8, 4, 2, 1):
        a = a + _shuffle(a, lanes ^ sh)
    return a


@functools.partial(
    pl.kernel,
    mesh=_mesh,
    out_type=jax.ShapeDtypeStruct((L,), jnp.float32),
    scratch_types=[
        pltpu.VMEM((PTS, D), jnp.float32),
        pltpu.VMEM((PTS, D), jnp.float32),
        pltpu.VMEM((PTS + L,), jnp.int32),
        pltpu.VMEM((2, PTS // 2), jnp.int32),
        pltpu.VMEM((CP + NS, D), jnp.float32),
        pltpu.VMEM((CP2,), jnp.float32),
        pltpu.VMEM((4, CP2), jnp.float32),
        pltpu.VMEM((2, CP2), jnp.float32),
        pltpu.VMEM((2, CP2), jnp.float32),
        pltpu.VMEM((NS, CP2), jnp.float32),
        pltpu.VMEM((L,), jnp.float32),
        pltpu.VMEM_SHARED((CP + NS, D), jnp.float32),
        pltpu.VMEM_SHARED((NS, CP2), jnp.float32),
        pltpu.VMEM_SHARED((NS, CP2), jnp.float32),
        pltpu.VMEM_SHARED((NS, CP2), jnp.float32),
    ],
)
def _loss_kernel(feat_hbm, lab_hbm, out_hbm,
                 feat_v, mrow_v, lab_v, lab2_v, sums_v, cnt_v, cnt4_v,
                 maxd_v, mind_v,
                 stage_v, out_v,
                 sh_sums, sh_cnt, sh_maxd, sh_mind):
    cid = lax.axis_index("c")
    sid = lax.axis_index("s")
    base = sid * PTS
    crow = sid * CPT

    zero16 = jnp.zeros((L,), jnp.float32)
    one16 = jnp.full((L,), 1.0, jnp.float32)
    lane0 = lax.iota(jnp.int32, L) == 0

    pltpu.sync_copy(feat_hbm.at[pl.ds(base, PTS), :], feat_v)
    pltpu.sync_copy(lab_hbm.at[pl.ds(base, PTS)], lab_v.at[pl.ds(0, PTS)])
    for h in range(2):
        pltpu.sync_copy(lab_hbm.at[pl.ds(base + h * (PTS // 2), PTS // 2)],
                        lab2_v.at[h])

    for r in range(CPT):
        for j in range(DCH):
            sums_v[r, pl.ds(j * L, L)] = zero16
    pltpu.sync_copy(sums_v.at[pl.ds(0, CPT), :],
                    sh_sums.at[pl.ds(crow, CPT), :])

    for k in range(CP2 // L):
        for q in range(4):
            cnt4_v[q, pl.ds(k * L, L)] = zero16
        for q in range(2):
            maxd_v[q, pl.ds(k * L, L)] = jnp.full((L,), -1e30, jnp.float32)
            mind_v[q, pl.ds(k * L, L)] = jnp.full((L,), 1e30, jnp.float32)

    onehot0 = jnp.where(lane0, 1.0, 0.0).astype(jnp.float32)

    def hist_body(i, _):
        for q in range(4):
            lab = lab_v[pl.ds(i * 4 + q, L)][0]
            win = cnt4_v[q, pl.ds(lab, L)]
            cnt4_v[q, pl.ds(lab, L)] = win + onehot0
        return 0
    lax.fori_loop(0, PTS // 4, hist_body, 0, unroll=4)
    for k in range(CP2 // L):
        cnt4_v[0, pl.ds(k * L, L)] = (
            (cnt4_v[0, pl.ds(k * L, L)] + cnt4_v[1, pl.ds(k * L, L)])
            + (cnt4_v[2, pl.ds(k * L, L)] + cnt4_v[3, pl.ds(k * L, L)]))
    pltpu.sync_copy(cnt4_v.at[0], sh_cnt.at[sid])

    plsc.subcore_barrier()

    half = PTS // 2
    for h in range(2):
        pltpu.sync_copy(feat_v.at[pl.ds(h * half, half), :],
                        sh_sums.at[lab2_v.at[h]], add=True)

    plsc.subcore_barrier()

    pltpu.sync_copy(sh_sums.at[pl.ds(crow, CPT), :],
                    sums_v.at[pl.ds(crow, CPT), :])
    pltpu.sync_copy(sh_cnt, stage_v)

    for k in range(CP2 // L):
        acc = zero16
        for t in range(NS):
            acc = acc + stage_v[t, pl.ds(k * L, L)]
        cnt_v[pl.ds(k * L, L)] = acc

    gacc = [zero16] * DCH
    s2acc = zero16
    for r in range(CPT):
        cc = crow + r
        cntv = jnp.full((L,), cnt_v[pl.ds(cc, L)][0], jnp.float32)
        invv = 1.0 / jnp.where(cntv > 0.0, cntv, 1.0)
        for j in range(DCH):
            srow = sums_v[cc, pl.ds(j * L, L)]
            gacc[j] = gacc[j] + srow
            m = srow * invv
            sums_v[cc, pl.ds(j * L, L)] = m
            s2acc = s2acc + cntv * (m * m)
    gr = CP + sid
    for j in range(DCH):
        sums_v[gr, pl.ds(j * L, L)] = gacc[j]
    mind_v[0, pl.ds(CP, L)] = _vsum(s2acc)
    pltpu.sync_copy(sums_v.at[pl.ds(crow, CPT), :],
                    sh_sums.at[pl.ds(crow, CPT), :])
    pltpu.sync_copy(sums_v.at[gr], sh_sums.at[gr])

    plsc.subcore_barrier()

    for h in range(2):
        pltpu.sync_copy(sh_sums.at[lab2_v.at[h]],
                        mrow_v.at[pl.ds(h * half, half), :])

    def dist_body(i, w):
        wout = []
        for q in range(2):
            p = i * 2 + q
            lab = lab_v[pl.ds(p, L)][0]
            acc0 = zero16
            acc1 = zero16
            for j in range(0, DCH, 2):
                df0 = feat_v[p, pl.ds(j * L, L)] - mrow_v[p, pl.ds(j * L, L)]
                acc0 = acc0 + df0 * df0
                df1 = (feat_v[p, pl.ds((j + 1) * L, L)]
                       - mrow_v[p, pl.ds((j + 1) * L, L)])
                acc1 = acc1 + df1 * df1
            dist = _vsum(acc0 + acc1)[0]
            winx = maxd_v[q, pl.ds(lab, L)]
            maxd_v[q, pl.ds(lab, L)] = jnp.where(
                lane0, jnp.maximum(winx, dist), winx)
            winn = mind_v[q, pl.ds(lab, L)]
            mind_v[q, pl.ds(lab, L)] = jnp.where(
                lane0, jnp.minimum(winn, dist), winn)
            wout.append(w[q] + dist)
        return tuple(wout)
    wp = lax.fori_loop(0, PTS // 2, dist_body,
                       (jnp.float32(0.0), jnp.float32(0.0)), unroll=4)
    w_part = wp[0] + wp[1]

    for k in range(CP2 // L):
        maxd_v[0, pl.ds(k * L, L)] = jnp.maximum(
            maxd_v[0, pl.ds(k * L, L)], maxd_v[1, pl.ds(k * L, L)])
        mind_v[0, pl.ds(k * L, L)] = jnp.minimum(
            mind_v[0, pl.ds(k * L, L)], mind_v[1, pl.ds(k * L, L)])
    maxd_v[0, pl.ds(CP, L)] = jnp.full((L,), w_part, jnp.float32)
    pltpu.sync_copy(maxd_v.at[0], sh_maxd.at[sid])
    pltpu.sync_copy(mind_v.at[0], sh_mind.at[sid])

    plsc.subcore_barrier()

    @pl.when(jnp.logical_and(sid == 0, cid == 0))
    def _final():
        pltpu.sync_copy(sh_maxd, stage_v)
        wacc = zero16
        for t in range(NS):
            wacc = wacc + stage_v[t, pl.ds(CP, L)]
        Wv = wacc

        for k in range(CP2 // L):
            acc = jnp.full((L,), -1e30, jnp.float32)
            for t in range(NS):
                acc = jnp.maximum(acc, stage_v[t, pl.ds(k * L, L)])
            maxd_v[0, pl.ds(k * L, L)] = acc
        pltpu.sync_copy(sh_mind, stage_v)
        s2v = zero16
        for t in range(NS):
            s2v = s2v + stage_v[t, pl.ds(CP, L)]
        for k in range(CP2 // L):
            acc = jnp.full((L,), 1e30, jnp.float32)
            for t in range(NS):
                acc = jnp.minimum(acc, stage_v[t, pl.ds(k * L, L)])
            mind_v[0, pl.ds(k * L, L)] = acc

        pltpu.sync_copy(sh_sums.at[pl.ds(CP, NS), :], stage_v)
        gsq = zero16
        for j in range(DCH):
            gj = zero16
            for t in range(NS):
                gj = gj + stage_v[t, pl.ds(j * L, L)]
            gj = gj * (1.0 / N)
            gsq = gsq + gj * gj
        Bv = s2v - N * _vsum(gsq)

        mm = zero16
        nu = zero16
        for k in range(CP2 // L):
            present = cnt_v[pl.ds(k * L, L)] > 0.0
            diff = maxd_v[0, pl.ds(k * L, L)] - mind_v[0, pl.ds(k * L, L)]
            mm = mm + jnp.where(present, diff, zero16)
            nu = nu + jnp.where(present, one16, zero16)
        mmv = _vsum(mm)
        nuv = _vsum(nu)

        totalv = WB_W * (Wv / (Bv + 1e-8)) + MM_W * (mmv / nuv)
        out_v[pl.ds(0, L)] = totalv
        pltpu.sync_copy(out_v, out_hbm)


def kernel(features, labels):
    labels = labels.astype(jnp.int32)
    out = _loss_kernel(features, labels)
    return out[0]


# speedup vs baseline: 1.0444x; 1.0444x over previous
import functools

import jax
import jax.numpy as jnp
from jax import lax
from jax.experimental import pallas as pl
from jax.experimental.pallas import tpu as pltpu
from jax.experimental.pallas import tpu_sc as plsc

N = 4096
D = 128
C = 100
CP = 112
CP2 = 128
NS = 16
PTS = N // NS
CPT = CP // NS
L = 16
DCH = D // L
WB_W = 1.0
MM_W = 0.1

_mesh = plsc.VectorSubcoreMesh(core_axis_name="c", subcore_axis_name="s",
                               num_cores=1)

_GATHER_DNUMS = lax.GatherDimensionNumbers(
    offset_dims=(), collapsed_slice_dims=(0,), start_index_map=(0,))


def _shuffle(a, perm):
    return lax.gather(a, perm[:, None], dimension_numbers=_GATHER_DNUMS,
                      slice_sizes=(1,),
                      mode=lax.GatherScatterMode.PROMISE_IN_BOUNDS)


def _vsum(a):
    lanes = lax.iota(jnp.int32, L)
    for sh in (8, 4, 2, 1):
        a = a + _shuffle(a, lanes ^ sh)
    return a


@functools.partial(
    pl.kernel,
    mesh=_mesh,
    out_type=jax.ShapeDtypeStruct((L,), jnp.float32),
    scratch_types=[
        pltpu.VMEM((PTS, D), jnp.float32),
        pltpu.VMEM((PTS, D), jnp.float32),
        pltpu.VMEM((PTS + L,), jnp.int32),
        pltpu.VMEM((2, PTS // 2), jnp.int32),
        pltpu.VMEM((CP + NS, D), jnp.float32),
        pltpu.VMEM((CP2,), jnp.float32),
        pltpu.VMEM((4, CP2), jnp.float32),
        pltpu.VMEM((2, CP2), jnp.float32),
        pltpu.VMEM((2, CP2), jnp.float32),
        pltpu.VMEM((NS, CP2), jnp.float32),
        pltpu.VMEM((L,), jnp.float32),
        pltpu.VMEM_SHARED((CP + NS, D), jnp.float32),
        pltpu.VMEM_SHARED((NS, CP2), jnp.float32),
        pltpu.VMEM_SHARED((NS, CP2), jnp.float32),
        pltpu.VMEM_SHARED((NS, CP2), jnp.float32),
        pltpu.SemaphoreType.DMA,
    ],
)
def _loss_kernel(feat_hbm, lab_hbm, out_hbm,
                 feat_v, mrow_v, lab_v, lab2_v, sums_v, cnt_v, cnt4_v,
                 maxd_v, mind_v,
                 stage_v, out_v,
                 sh_sums, sh_cnt, sh_maxd, sh_mind, sem):
    cid = lax.axis_index("c")
    sid = lax.axis_index("s")
    base = sid * PTS
    crow = sid * CPT

    zero16 = jnp.zeros((L,), jnp.float32)
    one16 = jnp.full((L,), 1.0, jnp.float32)
    lane0 = lax.iota(jnp.int32, L) == 0

    d1 = pltpu.async_copy(feat_hbm.at[pl.ds(base, PTS), :], feat_v, sem)
    d2 = pltpu.async_copy(lab_hbm.at[pl.ds(base, PTS)],
                          lab_v.at[pl.ds(0, PTS)], sem)
    dls = [pltpu.async_copy(
        lab_hbm.at[pl.ds(base + h * (PTS // 2), PTS // 2)],
        lab2_v.at[h], sem) for h in range(2)]

    for r in range(CPT):
        for j in range(DCH):
            sums_v[r, pl.ds(j * L, L)] = zero16
    pltpu.sync_copy(sums_v.at[pl.ds(0, CPT), :],
                    sh_sums.at[pl.ds(crow, CPT), :])

    for k in range(CP2 // L):
        for q in range(4):
            cnt4_v[q, pl.ds(k * L, L)] = zero16
        for q in range(2):
            maxd_v[q, pl.ds(k * L, L)] = jnp.full((L,), -1e30, jnp.float32)
            mind_v[q, pl.ds(k * L, L)] = jnp.full((L,), 1e30, jnp.float32)

    onehot0 = jnp.where(lane0, 1.0, 0.0).astype(jnp.float32)
    d2.wait()
    for d in dls:
        d.wait()
    d1.wait()

    def hist_body(i, _):
        for q in range(4):
            lab = lab_v[pl.ds(i * 4 + q, L)][0]
            win = cnt4_v[q, pl.ds(lab, L)]
            cnt4_v[q, pl.ds(lab, L)] = win + onehot0
        return 0
    lax.fori_loop(0, PTS // 4, hist_body, 0, unroll=4)
    for k in range(CP2 // L):
        cnt4_v[0, pl.ds(k * L, L)] = (
            (cnt4_v[0, pl.ds(k * L, L)] + cnt4_v[1, pl.ds(k * L, L)])
            + (cnt4_v[2, pl.ds(k * L, L)] + cnt4_v[3, pl.ds(k * L, L)]))
    pltpu.sync_copy(cnt4_v.at[0], sh_cnt.at[sid])

    plsc.subcore_barrier()

    half = PTS // 2
    for h in range(2):
        pltpu.sync_copy(feat_v.at[pl.ds(h * half, half), :],
                        sh_sums.at[lab2_v.at[h]], add=True)

    plsc.subcore_barrier()

    b1 = pltpu.async_copy(sh_sums.at[pl.ds(crow, CPT), :],
                          sums_v.at[pl.ds(crow, CPT), :], sem)
    b2 = pltpu.async_copy(sh_cnt, stage_v, sem)
    b1.wait()
    b2.wait()

    for k in range(CP2 // L):
        acc = zero16
        for t in range(NS):
            acc = acc + stage_v[t, pl.ds(k * L, L)]
        cnt_v[pl.ds(k * L, L)] = acc

    gacc = [zero16] * DCH
    s2acc = zero16
    for r in range(CPT):
        cc = crow + r
        cntv = jnp.full((L,), cnt_v[pl.ds(cc, L)][0], jnp.float32)
        invv = 1.0 / jnp.where(cntv > 0.0, cntv, 1.0)
        for j in range(DCH):
            srow = sums_v[cc, pl.ds(j * L, L)]
            gacc[j] = gacc[j] + srow
            m = srow * invv
            sums_v[cc, pl.ds(j * L, L)] = m
            s2acc = s2acc + cntv * (m * m)
    gr = CP + sid
    for j in range(DCH):
        sums_v[gr, pl.ds(j * L, L)] = gacc[j]
    mind_v[0, pl.ds(CP, L)] = _vsum(s2acc)
    pltpu.sync_copy(sums_v.at[pl.ds(crow, CPT), :],
                    sh_sums.at[pl.ds(crow, CPT), :])
    pltpu.sync_copy(sums_v.at[gr], sh_sums.at[gr])

    plsc.subcore_barrier()

    gds = [pltpu.async_copy(sh_sums.at[lab2_v.at[h]],
                            mrow_v.at[pl.ds(h * half, half), :], sem)
           for h in range(2)]
    for d in gds:
        d.wait()

    def dist_body(i, w):
        wout = []
        for q in range(2):
            p = i * 2 + q
            lab = lab_v[pl.ds(p, L)][0]
            acc0 = zero16
            acc1 = zero16
            for j in range(0, DCH, 2):
                df0 = feat_v[p, pl.ds(j * L, L)] - mrow_v[p, pl.ds(j * L, L)]
                acc0 = acc0 + df0 * df0
                df1 = (feat_v[p, pl.ds((j + 1) * L, L)]
                       - mrow_v[p, pl.ds((j + 1) * L, L)])
                acc1 = acc1 + df1 * df1
            dist = _vsum(acc0 + acc1)[0]
            winx = maxd_v[q, pl.ds(lab, L)]
            maxd_v[q, pl.ds(lab, L)] = jnp.where(
                lane0, jnp.maximum(winx, dist), winx)
            winn = mind_v[q, pl.ds(lab, L)]
            mind_v[q, pl.ds(lab, L)] = jnp.where(
                lane0, jnp.minimum(winn, dist), winn)
            wout.append(w[q] + dist)
        return tuple(wout)
    wp = lax.fori_loop(0, PTS // 2, dist_body,
                       (jnp.float32(0.0), jnp.float32(0.0)), unroll=4)
    w_part = wp[0] + wp[1]

    for k in range(CP2 // L):
        maxd_v[0, pl.ds(k * L, L)] = jnp.maximum(
            maxd_v[0, pl.ds(k * L, L)], maxd_v[1, pl.ds(k * L, L)])
        mind_v[0, pl.ds(k * L, L)] = jnp.minimum(
            mind_v[0, pl.ds(k * L, L)], mind_v[1, pl.ds(k * L, L)])
    maxd_v[0, pl.ds(CP, L)] = jnp.full((L,), w_part, jnp.float32)
    pltpu.sync_copy(maxd_v.at[0], sh_maxd.at[sid])
    pltpu.sync_copy(mind_v.at[0], sh_mind.at[sid])

    plsc.subcore_barrier()

    @pl.when(jnp.logical_and(sid == 0, cid == 0))
    def _final():
        pltpu.sync_copy(sh_maxd, stage_v)
        wacc = zero16
        for t in range(NS):
            wacc = wacc + stage_v[t, pl.ds(CP, L)]
        Wv = wacc

        for k in range(CP2 // L):
            acc = jnp.full((L,), -1e30, jnp.float32)
            for t in range(NS):
                acc = jnp.maximum(acc, stage_v[t, pl.ds(k * L, L)])
            maxd_v[0, pl.ds(k * L, L)] = acc
        pltpu.sync_copy(sh_mind, stage_v)
        s2v = zero16
        for t in range(NS):
            s2v = s2v + stage_v[t, pl.ds(CP, L)]
        for k in range(CP2 // L):
            acc = jnp.full((L,), 1e30, jnp.float32)
            for t in range(NS):
                acc = jnp.minimum(acc, stage_v[t, pl.ds(k * L, L)])
            mind_v[0, pl.ds(k * L, L)] = acc

        pltpu.sync_copy(sh_sums.at[pl.ds(CP, NS), :], stage_v)
        gsq = zero16
        for j in range(DCH):
            gj = zero16
            for t in range(NS):
                gj = gj + stage_v[t, pl.ds(j * L, L)]
            gj = gj * (1.0 / N)
            gsq = gsq + gj * gj
        Bv = s2v - N * _vsum(gsq)

        mm = zero16
        nu = zero16
        for k in range(CP2 // L):
            present = cnt_v[pl.ds(k * L, L)] > 0.0
            diff = maxd_v[0, pl.ds(k * L, L)] - mind_v[0, pl.ds(k * L, L)]
            mm = mm + jnp.where(present, diff, zero16)
            nu = nu + jnp.where(present, one16, zero16)
        mmv = _vsum(mm)
        nuv = _vsum(nu)

        totalv = WB_W * (Wv / (Bv + 1e-8)) + MM_W * (mmv / nuv)
        out_v[pl.ds(0, L)] = totalv
        pltpu.sync_copy(out_v, out_hbm)


def kernel(features, labels):
    labels = labels.astype(jnp.int32)
    out = _loss_kernel(features, labels)
    return out[0]
